# baseline (device time: 28711 ns/iter reference)
import jax
import jax.numpy as jnp
from jax import lax
from jax.experimental import pallas as pl
from jax.experimental.pallas import tpu as pltpu

Y_SIZE = 2
EPS = 1e-5

BLK = 512


def _comm_sums(x):
    m, n = x.shape
    nblk = m // BLK

    def body(x_hbm, inv_ref, xv, send_buf, in_sems, send_sem, recv_sem):
        my_x = lax.axis_index("x")
        my_y = lax.axis_index("y")
        peer_id = (my_x, 1 - my_y)

        barrier_sem = pltpu.get_barrier_semaphore()
        pl.semaphore_signal(
            barrier_sem, inc=1, device_id=peer_id,
            device_id_type=pl.DeviceIdType.MESH,
        )

        def rows(b):
            return pl.ds(b * BLK, BLK)

        copies = []
        for b in range(nblk):
            cp = pltpu.make_async_copy(
                x_hbm.at[rows(b), :], xv.at[rows(b), :], in_sems.at[b]
            )
            cp.start()
            copies.append(cp)

        ones_row = jnp.ones((1, n), dtype=jnp.float32)
        for b in range(nblk):
            copies[b].wait()
            blk = xv[rows(b), :]
            xsq = blk * blk
            sums = lax.dot_general(
                ones_row, xsq,
                dimension_numbers=(((1,), (1,)), ((), ())),
                preferred_element_type=jnp.float32,
            )
            send_buf[pl.ds(b, 1), :] = sums

        pl.semaphore_wait(barrier_sem, 1)

        rdma = pltpu.make_async_remote_copy(
            src_ref=send_buf,
            dst_ref=inv_ref,
            send_sem=send_sem,
            recv_sem=recv_sem,
            device_id=peer_id,
            device_id_type=pl.DeviceIdType.MESH,
        )
        rdma.start()
        rdma.wait()

        inv_ref[:, :] = lax.rsqrt(
            (send_buf[:, :] + inv_ref[:, :]) / (Y_SIZE * n) + EPS
        )

    return pl.pallas_call(
        body,
        out_shape=(
            jax.ShapeDtypeStruct((m // BLK, BLK), jnp.float32),
            jax.ShapeDtypeStruct((m, n), jnp.float32),
        ),
        in_specs=[pl.BlockSpec(memory_space=pl.ANY)],
        out_specs=(
            pl.BlockSpec(memory_space=pltpu.VMEM),
            pl.BlockSpec(memory_space=pltpu.VMEM),
        ),
        scratch_shapes=[
            pltpu.VMEM((m // BLK, BLK), jnp.float32),
            pltpu.SemaphoreType.DMA((m // BLK,)),
            pltpu.SemaphoreType.DMA,
            pltpu.SemaphoreType.DMA,
        ],
        compiler_params=pltpu.CompilerParams(
            collective_id=0,
            vmem_limit_bytes=96 * 1024 * 1024,
        ),
    )(x)


def _scale(xv, gamma2, inv8):
    m, n = xv.shape
    nblk = m // BLK

    def body(xv_ref, g_ref, inv_ref, out_hbm, ob, out_sems):
        def rows(b):
            return pl.ds(b * BLK, BLK)

        eye = (
            lax.broadcasted_iota(jnp.int32, (BLK, BLK), 0)
            == lax.broadcasted_iota(jnp.int32, (BLK, BLK), 1)
        ).astype(jnp.float32)

        out_copies = []
        for b in range(nblk):
            slot = b % 2
            if b >= 2:
                out_copies[b - 2].wait()
            inv_col = lax.dot_general(
                eye, inv_ref[b : b + 1, :],
                dimension_numbers=(((1,), (1,)), ((), ())),
                preferred_element_type=jnp.float32,
            )
            ob[slot, :, :] = xv_ref[rows(b), :] * g_ref[:, :] * inv_col
            cp = pltpu.make_async_copy(
                ob.at[slot], out_hbm.at[rows(b), :], out_sems.at[b]
            )
            cp.start()
            out_copies.append(cp)
        for b in range(max(nblk - 2, 0), nblk):
            out_copies[b].wait()

    return pl.pallas_call(
        body,
        out_shape=jax.ShapeDtypeStruct((m, n), xv.dtype),
        in_specs=[
            pl.BlockSpec(memory_space=pltpu.VMEM),
            pl.BlockSpec(memory_space=pltpu.VMEM),
            pl.BlockSpec(memory_space=pltpu.VMEM),
        ],
        out_specs=pl.BlockSpec(memory_space=pl.ANY),
        scratch_shapes=[
            pltpu.VMEM((2, BLK, n), jnp.float32),
            pltpu.SemaphoreType.DMA((m // BLK,)),
        ],
        compiler_params=pltpu.CompilerParams(
            vmem_limit_bytes=96 * 1024 * 1024,
        ),
    )(xv, gamma2, inv8)


def kernel(x, gamma):
    n = x.shape[1]
    gamma2 = gamma.reshape(1, n)
    inv8, xv = _comm_sums(x)
    return _scale(xv, gamma2, inv8)


# device time: 26584 ns/iter; 1.0800x vs baseline; 1.0800x over previous
import jax
import jax.numpy as jnp
from jax import lax
from jax.experimental import pallas as pl
from jax.experimental.pallas import tpu as pltpu

Y_SIZE = 2
EPS = 1e-5

BLK = 512


def _comm_sums(x):
    m, n = x.shape
    nblk = m // BLK

    half = nblk // 2

    def body(x_hbm, inv_ref, xb, send_buf, in_sems, send_sems, recv_sems):
        my_x = lax.axis_index("x")
        my_y = lax.axis_index("y")
        peer_id = (my_x, 1 - my_y)

        barrier_sem = pltpu.get_barrier_semaphore()
        pl.semaphore_signal(
            barrier_sem, inc=1, device_id=peer_id,
            device_id_type=pl.DeviceIdType.MESH,
        )

        def rows(b):
            return pl.ds(b * BLK, BLK)

        def half_rdma(h):
            return pltpu.make_async_remote_copy(
                src_ref=send_buf.at[pl.ds(h * half, half), :],
                dst_ref=inv_ref.at[pl.ds(h * half, half), :],
                send_sem=send_sems.at[h],
                recv_sem=recv_sems.at[h],
                device_id=peer_id,
                device_id_type=pl.DeviceIdType.MESH,
            )

        copies = []
        for b in range(nblk):
            copies.append(
                pltpu.make_async_copy(
                    x_hbm.at[rows(b), :], xb.at[b % 2], in_sems.at[b]
                )
            )
        copies[0].start()
        copies[1].start()

        ones_row = jnp.ones((1, n), dtype=jnp.float32)
        rdma_a = None
        for b in range(nblk):
            copies[b].wait()
            blk = xb[b % 2]
            xsq = blk * blk
            sums = lax.dot_general(
                ones_row, xsq,
                dimension_numbers=(((1,), (1,)), ((), ())),
                preferred_element_type=jnp.float32,
            )
            send_buf[pl.ds(b, 1), :] = sums
            if b + 2 < nblk:
                copies[b + 2].start()
            if b == half - 1:
                pl.semaphore_wait(barrier_sem, 1)
                rdma_a = half_rdma(0)
                rdma_a.start()

        rdma_b = half_rdma(1)
        rdma_b.start()
        rdma_a.wait()
        rdma_b.wait()

        inv_ref[:, :] = lax.rsqrt(
            (send_buf[:, :] + inv_ref[:, :]) / (Y_SIZE * n) + EPS
        )

    return pl.pallas_call(
        body,
        out_shape=jax.ShapeDtypeStruct((m // BLK, BLK), jnp.float32),
        in_specs=[pl.BlockSpec(memory_space=pl.ANY)],
        out_specs=pl.BlockSpec(memory_space=pltpu.VMEM),
        scratch_shapes=[
            pltpu.VMEM((2, BLK, n), jnp.float32),
            pltpu.VMEM((m // BLK, BLK), jnp.float32),
            pltpu.SemaphoreType.DMA((m // BLK,)),
            pltpu.SemaphoreType.DMA((2,)),
            pltpu.SemaphoreType.DMA((2,)),
        ],
        compiler_params=pltpu.CompilerParams(
            collective_id=0,
            vmem_limit_bytes=64 * 1024 * 1024,
        ),
    )(x)


def _scale(x, gamma2, inv8):
    m, n = x.shape
    nblk = m // BLK

    def body(x_hbm, g_ref, inv_ref, out_hbm, xb, ob, in_sems, out_sems):
        def rows(b):
            return pl.ds(b * BLK, BLK)

        eye = (
            lax.broadcasted_iota(jnp.int32, (BLK, BLK), 0)
            == lax.broadcasted_iota(jnp.int32, (BLK, BLK), 1)
        ).astype(jnp.float32)

        in_copies = []
        for b in range(nblk):
            in_copies.append(
                pltpu.make_async_copy(
                    x_hbm.at[rows(b), :], xb.at[b % 2], in_sems.at[b]
                )
            )
        in_copies[0].start()
        in_copies[1].start()

        out_copies = []
        for b in range(nblk):
            slot = b % 2
            in_copies[b].wait()
            if b >= 2:
                out_copies[b - 2].wait()
            inv_col = lax.dot_general(
                eye, inv_ref[b : b + 1, :],
                dimension_numbers=(((1,), (1,)), ((), ())),
                preferred_element_type=jnp.float32,
            )
            ob[slot, :, :] = xb[slot] * g_ref[:, :] * inv_col
            cp = pltpu.make_async_copy(
                ob.at[slot], out_hbm.at[rows(b), :], out_sems.at[b]
            )
            cp.start()
            out_copies.append(cp)
            if b + 2 < nblk:
                in_copies[b + 2].start()
        for b in range(max(nblk - 2, 0), nblk):
            out_copies[b].wait()

    return pl.pallas_call(
        body,
        out_shape=jax.ShapeDtypeStruct((m, n), x.dtype),
        in_specs=[
            pl.BlockSpec(memory_space=pl.ANY),
            pl.BlockSpec(memory_space=pltpu.VMEM),
            pl.BlockSpec(memory_space=pltpu.VMEM),
        ],
        out_specs=pl.BlockSpec(memory_space=pl.ANY),
        scratch_shapes=[
            pltpu.VMEM((2, BLK, n), jnp.float32),
            pltpu.VMEM((2, BLK, n), jnp.float32),
            pltpu.SemaphoreType.DMA((m // BLK,)),
            pltpu.SemaphoreType.DMA((m // BLK,)),
        ],
        compiler_params=pltpu.CompilerParams(
            vmem_limit_bytes=64 * 1024 * 1024,
        ),
    )(x, gamma2, inv8)


def kernel(x, gamma):
    n = x.shape[1]
    gamma2 = gamma.reshape(1, n)
    inv8 = _comm_sums(x)
    return _scale(x, gamma2, inv8)
